# baseline (speedup 1.0000x reference)
import functools

import jax
import jax.numpy as jnp
from jax import lax
from jax.experimental import pallas as pl
from jax.experimental.pallas import tpu as pltpu
from jax.experimental.pallas import tpu_sc as plsc

_NSC = 2
_CH = 128


@functools.lru_cache(maxsize=None)
def _build(n, d):
    rows_c = n // _NSC
    n_chunks = rows_c // _CH
    mesh = plsc.ScalarSubcoreMesh(axis_name="c", num_cores=_NSC)

    @functools.partial(
        pl.kernel,
        mesh=mesh,
        out_type=jax.ShapeDtypeStruct((n, d), jnp.float32),
        scratch_types=[
            pltpu.VMEM_SHARED((rows_c, d), jnp.float32),
            pltpu.VMEM_SHARED((1, d), jnp.float32),
            pltpu.SMEM((16,), jnp.int32),
            pltpu.SemaphoreType.DMA,
            pltpu.SemaphoreType.DMA,
            pltpu.SemaphoreType.DMA,
        ],
    )
    def k(table_hbm, clamp_hbm, out_hbm, buf, fixbuf, smem,
          isem, osem, csem):
        cid = lax.axis_index("c")
        base = cid * rows_c
        cc = pltpu.async_copy(clamp_hbm, smem, csem)
        ins = [
            pltpu.async_copy(
                table_hbm.at[pl.ds(base + j * _CH, _CH)],
                buf.at[pl.ds(j * _CH, _CH)],
                isem,
            )
            for j in range(n_chunks)
        ]
        for c in ins:
            c.wait()
        outs = [
            pltpu.async_copy(
                buf.at[pl.ds(j * _CH, _CH)],
                out_hbm.at[pl.ds(base + j * _CH, _CH)],
                osem,
            )
            for j in range(n_chunks)
        ]
        for c in outs:
            c.wait()
        cc.wait()
        clamp_s = smem[0]

        @pl.when(clamp_s < base + rows_c - 1)
        def _tail():
            pltpu.sync_copy(table_hbm.at[pl.ds(clamp_s, 1)], fixbuf)

            def _fix(r, carry):
                pltpu.sync_copy(fixbuf, out_hbm.at[pl.ds(r, 1)])
                return carry

            lax.fori_loop(jnp.maximum(clamp_s + 1, base), base + rows_c,
                          _fix, 0)

    return k


def kernel(seq_len, table):
    n, d = table.shape
    clamp_val = jnp.maximum(jnp.asarray(seq_len, jnp.int32) - 1, 0)
    clamp = jnp.broadcast_to(clamp_val, (16,))
    return _build(n, d)(table, clamp)


# speedup vs baseline: 1.0883x; 1.0883x over previous
import functools

import jax
import jax.numpy as jnp
from jax import lax
from jax.experimental import pallas as pl
from jax.experimental.pallas import tpu as pltpu
from jax.experimental.pallas import tpu_sc as plsc

_NSC = 2
_CH = 256


@functools.lru_cache(maxsize=None)
def _build(n, d):
    rows_c = n // _NSC
    n_chunks = rows_c // _CH
    mesh = plsc.ScalarSubcoreMesh(axis_name="c", num_cores=_NSC)

    @functools.partial(
        pl.kernel,
        mesh=mesh,
        out_type=jax.ShapeDtypeStruct((n, d), jnp.float32),
        scratch_types=[
            pltpu.VMEM_SHARED((rows_c, d), jnp.float32),
            pltpu.VMEM_SHARED((1, d), jnp.float32),
            pltpu.SMEM((16,), jnp.int32),
            pltpu.SemaphoreType.DMA((n_chunks,)),
            pltpu.SemaphoreType.DMA,
            pltpu.SemaphoreType.DMA,
        ],
    )
    def k(table_hbm, clamp_hbm, out_hbm, buf, fixbuf, smem,
          isems, osem, csem):
        cid = lax.axis_index("c")
        base = cid * rows_c
        cc = pltpu.async_copy(clamp_hbm, smem, csem)
        ins = [
            pltpu.async_copy(
                table_hbm.at[pl.ds(base + j * _CH, _CH)],
                buf.at[pl.ds(j * _CH, _CH)],
                isems.at[j],
            )
            for j in range(n_chunks)
        ]
        outs = []
        for j in range(n_chunks):
            ins[j].wait()
            outs.append(pltpu.async_copy(
                buf.at[pl.ds(j * _CH, _CH)],
                out_hbm.at[pl.ds(base + j * _CH, _CH)],
                osem,
            ))
        for c in outs:
            c.wait()
        cc.wait()
        clamp_s = smem[0]

        @pl.when(clamp_s < base + rows_c - 1)
        def _tail():
            pltpu.sync_copy(table_hbm.at[pl.ds(clamp_s, 1)], fixbuf)

            def _fix(r, carry):
                pltpu.sync_copy(fixbuf, out_hbm.at[pl.ds(r, 1)])
                return carry

            lax.fori_loop(jnp.maximum(clamp_s + 1, base), base + rows_c,
                          _fix, 0)

    return k


def kernel(seq_len, table):
    n, d = table.shape
    clamp_val = jnp.maximum(jnp.asarray(seq_len, jnp.int32) - 1, 0)
    clamp = jnp.broadcast_to(clamp_val, (16,))
    return _build(n, d)(table, clamp)
